# baseline (speedup 1.0000x reference)
import functools

import jax
import jax.numpy as jnp
from jax import lax
from jax.experimental import pallas as pl
from jax.experimental.pallas import tpu as pltpu
from jax.experimental.pallas import tpu_sc as plsc

B, L, D = 16384, 200, 128
NC, NS = 2, 16
NW = NC * NS
RPW = B // NW
XBLK = 64
ROW_W = D * L
PAIRS_PER_XBLK = XBLK // 2
NDCH = D // 16


N_LG = 13


def _compute_row(xbuf, xoff, w0v, dwv, obuf, obase):
    w0s = tuple(w0v[pl.ds(k * 16, 16)] for k in range(NDCH))
    dws = tuple(dwv[pl.ds(k * 16, 16)] for k in range(NDCH))

    @plsc.parallel_loop(0, N_LG, step=1, unroll=8, carry=(w0s, dws))
    def gbody(lg, c):
        w0c, dwc = c
        lstart = jnp.minimum(lg * 16, L - 16)
        xc = xbuf[pl.ds(xoff + lstart, 16)].astype(jnp.float32)
        ob = obase + lstart * D
        for j in range(16):
            xf = jnp.full((16,), xc[j], jnp.float32)
            o = ob + j * D
            for k in range(NDCH):
                obuf[pl.ds(o + k * 16, 16)] = w0c[k] + dwc[k] * xf
        return c


NSLOT = 2
RPS = 1
RPI = NSLOT * RPS
QUADS_PER_XBLK = XBLK // RPI
assert RPW % RPI == 0 and XBLK % RPI == 0 and RPW % XBLK == 0


NXB = RPW // XBLK


def _sc_body(x_hbm, w0_hbm, dw_hbm, out_hbm, xbuf, obuf, w0v, dwv,
             xsem0, xsem1, *sems):
    wid = lax.axis_index("s") * NC + lax.axis_index("c")
    base_row = wid * RPW
    pltpu.sync_copy(w0_hbm, w0v)
    pltpu.sync_copy(dw_hbm, dwv)
    xsems = (xsem0, xsem1)

    def _x_dma(blk, slot):
        return pltpu.make_async_copy(
            x_hbm.at[pl.ds((base_row + blk * XBLK) * L, XBLK * L)],
            xbuf.at[pl.ds(slot * XBLK * L, XBLK * L)], xsems[slot])

    _x_dma(0, 0).start()

    def quad_body(q, _):
        row0 = base_row + RPI * q
        blk = q // QUADS_PER_XBLK
        xslot = blk % 2

        @pl.when(q % QUADS_PER_XBLK == 0)
        def _stage_x():
            for s in (0, 1):
                @pl.when((blk + 1 < NXB) & (xslot == 1 - s))
                def _refill(s=s):
                    _x_dma(blk + 1, s).start()

                @pl.when(xslot == s)
                def _wait_x(s=s):
                    _x_dma(blk, s).wait()

        xoff0 = (xslot * XBLK + (q % QUADS_PER_XBLK) * RPI) * L
        for s in range(NSLOT):
            @pl.when(q >= 1)
            def _wait(s=s):
                pltpu.make_async_copy(
                    obuf.at[pl.ds(s * RPS * ROW_W, RPS * ROW_W)],
                    out_hbm.at[pl.ds((row0 + s * RPS - RPI) * ROW_W,
                                     RPS * ROW_W)],
                    sems[s]).wait()

            for r in range(RPS):
                _compute_row(xbuf, xoff0 + (s * RPS + r) * L, w0v, dwv,
                             obuf, (s * RPS + r) * ROW_W)
            pltpu.make_async_copy(
                obuf.at[pl.ds(s * RPS * ROW_W, RPS * ROW_W)],
                out_hbm.at[pl.ds((row0 + s * RPS) * ROW_W, RPS * ROW_W)],
                sems[s]).start()
        return 0

    lax.fori_loop(0, RPW // RPI, quad_body, 0)
    last = base_row + RPW
    for s in range(NSLOT):
        pltpu.make_async_copy(
            obuf.at[pl.ds(s * RPS * ROW_W, RPS * ROW_W)],
            out_hbm.at[pl.ds((last + s * RPS - RPI) * ROW_W, RPS * ROW_W)],
            sems[s]).wait()


_sc_call = functools.partial(
    pl.kernel,
    out_type=jax.ShapeDtypeStruct((B * ROW_W,), jnp.float32),
    mesh=plsc.VectorSubcoreMesh(core_axis_name="c", subcore_axis_name="s"),
    scratch_types=[
        pltpu.VMEM((2 * XBLK * L,), jnp.int32),
        pltpu.VMEM((NSLOT * RPS * ROW_W,), jnp.float32),
        pltpu.VMEM((D,), jnp.float32),
        pltpu.VMEM((D,), jnp.float32),
    ] + [pltpu.SemaphoreType.DMA] * (2 + NSLOT),
)(_sc_body)


def kernel(x, embedding_weight):
    x = x.astype(jnp.int32).reshape(-1)
    w0 = embedding_weight[0]
    dw = embedding_weight[1] - embedding_weight[0]
    out = _sc_call(x, w0, dw)
    return jnp.transpose(out.reshape(B, L, D), (0, 2, 1))


# speedup vs baseline: 1.8810x; 1.8810x over previous
import functools

import jax
import jax.numpy as jnp
from jax import lax
from jax.experimental import pallas as pl
from jax.experimental.pallas import tpu as pltpu
from jax.experimental.pallas import tpu_sc as plsc

B, L, D = 16384, 200, 128
NC, NS = 2, 16
NW = NC * NS
RPW = B // NW
XBLK = 64
ROW_W = D * L
PAIRS_PER_XBLK = XBLK // 2
NDCH = D // 16


N_LG = 13


def _compute_row(xbuf, xoff, w0v, dwv, obuf, obase):
    w0s = tuple(w0v[pl.ds(k * 16, 16)] for k in range(NDCH))
    dws = tuple(dwv[pl.ds(k * 16, 16)] for k in range(NDCH))

    @plsc.parallel_loop(0, N_LG, step=1, unroll=4, carry=(w0s, dws))
    def gbody(lg, c):
        w0c, dwc = c
        lstart = jnp.minimum(lg * 16, L - 16)
        xc = xbuf[pl.ds(xoff + lstart, 16)].astype(jnp.float32)
        ob = obase + lstart * D
        for j in range(16):
            xf = jnp.full((16,), xc[j], jnp.float32)
            o = ob + j * D
            for k in range(NDCH):
                obuf[pl.ds(o + k * 16, 16)] = w0c[k] + dwc[k] * xf
        return c


NSLOT = 2
RPS = 1
RPI = NSLOT * RPS
QUADS_PER_XBLK = XBLK // RPI
assert RPW % RPI == 0 and XBLK % RPI == 0 and RPW % XBLK == 0


NXB = RPW // XBLK


def _sc_body(x_hbm, w0_hbm, dw_hbm, out_hbm, xbuf, obuf, w0v, dwv,
             xsem0, xsem1, *sems):
    wid = lax.axis_index("s") * NC + lax.axis_index("c")
    base_row = wid * RPW
    pltpu.sync_copy(w0_hbm, w0v)
    pltpu.sync_copy(dw_hbm, dwv)
    xsems = (xsem0, xsem1)

    def _x_dma(blk, slot):
        return pltpu.make_async_copy(
            x_hbm.at[pl.ds((base_row + blk * XBLK) * L, XBLK * L)],
            xbuf.at[pl.ds(slot * XBLK * L, XBLK * L)], xsems[slot])

    _x_dma(0, 0).start()

    def quad_body(q, _):
        row0 = base_row + RPI * q
        blk = q // QUADS_PER_XBLK
        xslot = blk % 2

        @pl.when(q % QUADS_PER_XBLK == 0)
        def _stage_x():
            for s in (0, 1):
                @pl.when((blk + 1 < NXB) & (xslot == 1 - s))
                def _refill(s=s):
                    _x_dma(blk + 1, s).start()

                @pl.when(xslot == s)
                def _wait_x(s=s):
                    _x_dma(blk, s).wait()

        xoff0 = (xslot * XBLK + (q % QUADS_PER_XBLK) * RPI) * L
        for s in range(NSLOT):
            @pl.when(q >= 1)
            def _wait(s=s):
                pltpu.make_async_copy(
                    obuf.at[pl.ds(s * RPS * ROW_W, RPS * ROW_W)],
                    out_hbm.at[pl.ds((row0 + s * RPS - RPI) * ROW_W,
                                     RPS * ROW_W)],
                    sems[s]).wait()

            for r in range(RPS):
                _compute_row(xbuf, xoff0 + (s * RPS + r) * L, w0v, dwv,
                             obuf, (s * RPS + r) * ROW_W)
            pltpu.make_async_copy(
                obuf.at[pl.ds(s * RPS * ROW_W, RPS * ROW_W)],
                out_hbm.at[pl.ds((row0 + s * RPS) * ROW_W, RPS * ROW_W)],
                sems[s]).start()
        return 0

    lax.fori_loop(0, RPW // RPI, quad_body, 0)
    last = base_row + RPW
    for s in range(NSLOT):
        pltpu.make_async_copy(
            obuf.at[pl.ds(s * RPS * ROW_W, RPS * ROW_W)],
            out_hbm.at[pl.ds((last + s * RPS - RPI) * ROW_W, RPS * ROW_W)],
            sems[s]).wait()


_sc_call = functools.partial(
    pl.kernel,
    out_type=jax.ShapeDtypeStruct((B * ROW_W,), jnp.float32),
    mesh=plsc.VectorSubcoreMesh(core_axis_name="c", subcore_axis_name="s"),
    scratch_types=[
        pltpu.VMEM((2 * XBLK * L,), jnp.int32),
        pltpu.VMEM((NSLOT * RPS * ROW_W,), jnp.float32),
        pltpu.VMEM((D,), jnp.float32),
        pltpu.VMEM((D,), jnp.float32),
    ] + [pltpu.SemaphoreType.DMA] * (2 + NSLOT),
)(_sc_body)


def kernel(x, embedding_weight):
    x = x.astype(jnp.int32).reshape(-1)
    w0 = embedding_weight[0]
    dw = embedding_weight[1] - embedding_weight[0]
    out = _sc_call(x, w0, dw)
    return jnp.transpose(out.reshape(B, L, D), (0, 2, 1))


# speedup vs baseline: 1.8812x; 1.0001x over previous
import functools

import jax
import jax.numpy as jnp
from jax import lax
from jax.experimental import pallas as pl
from jax.experimental.pallas import tpu as pltpu
from jax.experimental.pallas import tpu_sc as plsc

B, L, D = 16384, 200, 128
NC, NS = 2, 16
NW = NC * NS
RPW = B // NW
XBLK = 64
ROW_W = D * L
NDCH = D // 16


N_LG = 13


def _compute_row(xbuf, xoff, w0v, dwv, obuf, obase):
    w0s = tuple(w0v[pl.ds(k * 16, 16)] for k in range(NDCH))
    dws = tuple(dwv[pl.ds(k * 16, 16)] for k in range(NDCH))

    @plsc.parallel_loop(0, N_LG, step=1, unroll=4, carry=(w0s, dws))
    def gbody(lg, c):
        w0c, dwc = c
        lstart = jnp.minimum(lg * 16, L - 16)
        xc = xbuf[pl.ds(xoff + lstart, 16)].astype(jnp.float32)
        ob = obase + lstart * D
        for j in range(16):
            xf = jnp.full((16,), xc[j], jnp.float32)
            o = ob + j * D
            for k in range(NDCH):
                obuf[pl.ds(o + k * 16, 16)] = w0c[k] + dwc[k] * xf
        return c


NSLOT = 2
RPS = 1
RPI = NSLOT * RPS
QUADS_PER_XBLK = XBLK // RPI
assert RPW % RPI == 0 and XBLK % RPI == 0 and RPW % XBLK == 0


NXB = RPW // XBLK


def _sc_body(x_hbm, w0_hbm, dw_hbm, out_hbm, xbuf, obuf, w0v, dwv,
             xsem0, xsem1, *sems):
    wid = lax.axis_index("s") * NC + lax.axis_index("c")
    base_row = wid * RPW
    pltpu.sync_copy(w0_hbm, w0v)
    pltpu.sync_copy(dw_hbm, dwv)
    xsems = (xsem0, xsem1)

    def _x_dma(blk, slot):
        return pltpu.make_async_copy(
            x_hbm.at[pl.ds((base_row + blk * XBLK) * L, XBLK * L)],
            xbuf.at[pl.ds(slot * XBLK * L, XBLK * L)], xsems[slot])

    _x_dma(0, 0).start()

    def quad_body(q, _):
        row0 = base_row + RPI * q
        blk = q // QUADS_PER_XBLK
        xslot = blk % 2

        @pl.when(q % QUADS_PER_XBLK == 0)
        def _stage_x():
            for s in (0, 1):
                @pl.when((blk + 1 < NXB) & (xslot == 1 - s))
                def _refill(s=s):
                    _x_dma(blk + 1, s).start()

                @pl.when(xslot == s)
                def _wait_x(s=s):
                    _x_dma(blk, s).wait()

        xoff0 = (xslot * XBLK + (q % QUADS_PER_XBLK) * RPI) * L
        for s in range(NSLOT):
            @pl.when(q >= 1)
            def _wait(s=s):
                pltpu.make_async_copy(
                    obuf.at[pl.ds(s * RPS * ROW_W, RPS * ROW_W)],
                    out_hbm.at[pl.ds((row0 + s * RPS - RPI) * ROW_W,
                                     RPS * ROW_W)],
                    sems[s]).wait()

            for r in range(RPS):
                _compute_row(xbuf, xoff0 + (s * RPS + r) * L, w0v, dwv,
                             obuf, (s * RPS + r) * ROW_W)
            pltpu.make_async_copy(
                obuf.at[pl.ds(s * RPS * ROW_W, RPS * ROW_W)],
                out_hbm.at[pl.ds((row0 + s * RPS) * ROW_W, RPS * ROW_W)],
                sems[s]).start()
        return 0

    lax.fori_loop(0, RPW // RPI, quad_body, 0)
    last = base_row + RPW
    for s in range(NSLOT):
        pltpu.make_async_copy(
            obuf.at[pl.ds(s * RPS * ROW_W, RPS * ROW_W)],
            out_hbm.at[pl.ds((last + s * RPS - RPI) * ROW_W, RPS * ROW_W)],
            sems[s]).wait()


_sc_call = functools.partial(
    pl.kernel,
    out_type=jax.ShapeDtypeStruct((B * ROW_W,), jnp.float32),
    mesh=plsc.VectorSubcoreMesh(core_axis_name="c", subcore_axis_name="s"),
    scratch_types=[
        pltpu.VMEM((2 * XBLK * L,), jnp.int32),
        pltpu.VMEM((NSLOT * RPS * ROW_W,), jnp.float32),
        pltpu.VMEM((D,), jnp.float32),
        pltpu.VMEM((D,), jnp.float32),
    ] + [pltpu.SemaphoreType.DMA] * (2 + NSLOT),
)(_sc_body)


def kernel(x, embedding_weight):
    x = x.astype(jnp.int32).reshape(-1)
    w0 = embedding_weight[0]
    dw = embedding_weight[1] - embedding_weight[0]
    out = _sc_call(x, w0, dw)
    return jnp.transpose(out.reshape(B, L, D), (0, 2, 1))
